# baseline (speedup 1.0000x reference)
import functools

import jax
import jax.numpy as jnp
from jax import lax
from jax.experimental import pallas as pl
from jax.experimental.pallas import tpu as pltpu
from jax.experimental.pallas import tpu_sc as plsc

VOCAB = 1000000
DIM = 64
DP = DIM // 2
B = 16384
NEG = 20
B2 = 4096
L = 5
W = 10
NC, NS = 2, 16
NW = NC * NS
CG = 64
NROWS = CG * NEG
NTN = NROWS // 128
NCW = B // NW // CG
NCM = (B2 * W) // NW // CG
B2W_ = B2 // NW

_ILV = plsc.PackFormat.INTERLEAVED


def _bc(s, n=16):
    return lax.broadcast_in_dim(s, (n,), ())


@functools.partial(
    pl.kernel,
    out_type=(jax.ShapeDtypeStruct((B * NEG,), jnp.float32),
              jax.ShapeDtypeStruct((B,), jnp.float32),
              jax.ShapeDtypeStruct((B2 * W * NEG,), jnp.float32),
              jax.ShapeDtypeStruct((B2 * W,), jnp.float32)),
    mesh=plsc.VectorSubcoreMesh(core_axis_name="c", subcore_axis_name="s"),
    compiler_params=pltpu.CompilerParams(
        use_tc_tiling_on_sc=False, needs_layout_passes=False),
    scratch_types=[
        pltpu.VMEM((2, NROWS, DP), jnp.int32),
        pltpu.VMEM((2, CG, DP), jnp.int32),
        pltpu.VMEM((2, CG, DP), jnp.int32),
        pltpu.VMEM((B2W_, DP), jnp.int32),
        pltpu.VMEM((2, NTN, 128), jnp.int32),
        pltpu.VMEM((2, CG), jnp.int32),
        pltpu.VMEM((2, CG), jnp.int32),
        pltpu.VMEM((2, NROWS), jnp.float32),
        pltpu.VMEM((NCM * CG,), jnp.float32),
        pltpu.VMEM((B2W_,), jnp.int32),
        pltpu.SemaphoreType.DMA,
        pltpu.SemaphoreType.DMA,
        pltpu.SemaphoreType.DMA,
    ],
)
def _sc_dots(ct, xt, cw2, ow2, nw2, mw3, ml2, om2, nm2,
             dnw_out, dpw_out, dnm_out, dpm_out,
             nvm, pvm, cvm, mvm, nidx, pidx, cidx, dnvm, dp_all, lvm,
             sem_i, sem_g, sem_w):
    wid = lax.axis_index("s") * NC + lax.axis_index("c")
    iota = lax.iota(jnp.int32, 16)

    def unpk(w):
        return plsc.unpack(plsc.bitcast(w, jnp.bfloat16), format=_ILV,
                           preferred_element_type=jnp.float32)

    pltpu.sync_copy(mw3.at[wid], nidx.at[0, pl.ds(0, L)])
    pltpu.sync_copy(ml2.at[wid], lvm)
    cps = [pltpu.async_copy(ct.at[nidx.at[0, t]],
                            nvm.at[0, pl.ds(t * 128, 128)], sem_g)
           for t in range(L)]
    for cp in cps:
        cp.wait()
    for bb in range(B2W_ // 16):
        bv = bb * 16 + iota
        lnv = lvm[pl.ds(bb * 16, 16)]
        lnf = lnv.astype(jnp.float32)

        def mbody(p, _):
            col = _bc(p)
            acc_e = jnp.zeros((16,), jnp.float32)
            acc_o = jnp.zeros((16,), jnp.float32)
            for l in range(L):
                re, ro = unpk(plsc.load_gather(nvm, [_bc(0), bv * L + l, col]))
                m = jnp.full((16,), l, jnp.int32) < lnv
                acc_e = acc_e + jnp.where(m, re, 0.0)
                acc_o = acc_o + jnp.where(m, ro, 0.0)
            packed = plsc.bitcast(
                plsc.pack(acc_e / lnf, acc_o / lnf, format=_ILV), jnp.int32)
            plsc.store_scatter(mvm, [bv, col], packed)
            return 0
        lax.fori_loop(0, DP, mbody, 0)

    def run_phase(ncc, is_word):
        cbase = wid * ncc

        def idx_copies(c, buf):
            cglob = cbase + c
            ops = [pltpu.make_async_copy(
                (nw2 if is_word else nm2).at[pl.ds(cglob * NTN, NTN)],
                nidx.at[buf], sem_i)]
            prow, pcol = cglob // 2, (cglob % 2) * CG
            ops.append(pltpu.make_async_copy(
                (ow2 if is_word else om2).at[prow, pl.ds(pcol, CG)],
                pidx.at[buf], sem_i))
            if is_word:
                ops.append(pltpu.make_async_copy(
                    cw2.at[prow, pl.ds(pcol, CG)], cidx.at[buf], sem_i))
            return ops

        def row_gathers(c, buf):
            ops = [pltpu.make_async_copy(
                xt.at[nidx.at[buf, t]],
                nvm.at[buf, pl.ds(t * 128, 128)], sem_g)
                for t in range(NTN)]
            ops.append(pltpu.make_async_copy(
                xt.at[pidx.at[buf]], pvm.at[buf], sem_g))
            if is_word:
                ops.append(pltpu.make_async_copy(
                    ct.at[cidx.at[buf]], cvm.at[buf], sem_g))
            return ops

        def dn_writeback(c, buf):
            cglob = cbase + c
            return pltpu.make_async_copy(
                dnvm.at[buf],
                (dnw_out if is_word else dnm_out).at[pl.ds(cglob * NROWS, NROWS)],
                sem_w)

        for op in idx_copies(0, 0):
            op.start()
            op.wait()
        if ncc > 1:
            for op in idx_copies(1, 1):
                op.start()
                op.wait()
        for op in row_gathers(0, 0):
            op.start()

        def chunk_body(c, _):
            buf = lax.rem(c, 2)
            nbuf = lax.rem(c + 1, 2)

            @pl.when((c >= 1) & (c + 1 < ncc))
            def _():
                for op in idx_copies(c + 1, nbuf):
                    op.wait()

            @pl.when(c + 1 < ncc)
            def _():
                for op in row_gathers(c + 1, nbuf):
                    op.start()

            for op in row_gathers(c, buf):
                op.wait()

            @pl.when(c + 2 < ncc)
            def _():
                for op in idx_copies(c + 2, buf):
                    op.start()

            @pl.when(c >= 2)
            def _():
                dn_writeback(c - 2, buf).wait()

            for kk in range(CG // 16):
                g = kk * 16 + iota
                if is_word:
                    crow = g
                else:
                    crow = (c * CG + g) // W
                nbase = g * NEG
                bufv = _bc(buf)

                def dbody(p, accs):
                    col = _bc(p)
                    if is_word:
                        ce, co = unpk(plsc.load_gather(cvm, [bufv, crow, col]))
                    else:
                        ce, co = unpk(plsc.load_gather(mvm, [crow, col]))
                    new = []
                    for j in range(NEG):
                        xe, xo = unpk(
                            plsc.load_gather(nvm, [bufv, nbase + j, col]))
                        new.append(accs[j] + xe * ce + xo * co)
                    pe, po = unpk(plsc.load_gather(pvm, [bufv, g, col]))
                    new.append(accs[NEG] + pe * ce + po * co)
                    return tuple(new)
                accs = lax.fori_loop(
                    0, DP, dbody,
                    tuple(jnp.zeros((16,), jnp.float32) for _ in range(NEG + 1)))
                if is_word:
                    for j in range(NEG):
                        plsc.store_scatter(dnvm, [bufv, nbase + j], accs[j])
                    plsc.store_scatter(dp_all, [c * CG + g], -accs[NEG])
                else:
                    kval = plsc.load_gather(pidx, [bufv, g])
                    keep = kval != 0
                    neg_big = jnp.full((16,), -1e9, jnp.float32)
                    for j in range(NEG):
                        v = jnp.where(keep, accs[j], neg_big)
                        plsc.store_scatter(dnvm, [bufv, nbase + j], v)
                    vp = jnp.where(keep, -accs[NEG], neg_big)
                    plsc.store_scatter(dp_all, [c * CG + g], vp)
            dn_writeback(c, buf).start()
            return 0
        lax.fori_loop(0, ncc, chunk_body, 0)

        if ncc >= 2:
            dn_writeback(ncc - 2, (ncc - 2) % 2).wait()
        dn_writeback(ncc - 1, (ncc - 1) % 2).wait()
        pw_out = dpw_out if is_word else dpm_out
        pltpu.sync_copy(dp_all.at[pl.ds(0, ncc * CG)],
                        pw_out.at[pl.ds(cbase * CG, ncc * CG)])

    run_phase(NCW, True)
    run_phase(NCM, False)


def _tc_body(dnw_ref, dpw_ref, dnm_ref, dpm_ref, omw_ref, out_ref):
    def sp_sum(x):
        return jnp.sum(jnp.maximum(x, 0.0) + jnp.log(1.0 + jnp.exp(-jnp.abs(x))))
    lw = sp_sum(dnw_ref[...]) + sp_sum(dpw_ref[...])
    lm = sp_sum(dnm_ref[...]) + sp_sum(dpm_ref[...])
    cnt = jnp.sum((omw_ref[...] != 0).astype(jnp.float32))
    out_ref[...] = jnp.reshape(lw / B + 25.0 * lm / cnt, (1, 1))


def _packed(table):
    u = lax.bitcast_convert_type(table, jnp.uint32)

    def rbf(x):
        return (x + jnp.uint32(0x7FFF) + ((x >> 16) & jnp.uint32(1))) >> 16

    pk = rbf(u[:, 0::2]) | (rbf(u[:, 1::2]) << 16)
    pk = lax.bitcast_convert_type(pk, jnp.int32)
    pk = lax.optimization_barrier(pk.reshape(VOCAB // 4, 128))
    return pk.reshape(VOCAB, DP)


def kernel(center_words, outside_words, negative_examples_words, mwe_words,
           mwe_length, outside_mwe_words, negative_examples_mwe,
           center_table, context_table):
    cw2 = center_words.reshape(B // 128, 128)
    ow2 = outside_words.reshape(B // 128, 128)
    nw2 = negative_examples_words.reshape(B * NEG // 128, 128)
    mw3 = mwe_words.reshape(NW, L, 128)
    ml2 = mwe_length.reshape(NW, B2W_)
    om2 = outside_mwe_words.reshape(B2 * W // 128, 128)
    nm2 = negative_examples_mwe.reshape(B2 * W * NEG // 128, 128)

    dnw, dpw, dnm, dpm = _sc_dots(_packed(center_table), _packed(context_table),
                                  cw2, ow2, nw2, mw3, ml2, om2, nm2)

    out = pl.pallas_call(
        _tc_body,
        out_shape=jax.ShapeDtypeStruct((1, 1), jnp.float32),
    )(dnw.reshape(B * NEG // 128, 128),
      dpw.reshape(B // 128, 128),
      dnm.reshape(B2 * W * NEG // 128, 128),
      dpm.reshape(B2 * W // 128, 128),
      outside_mwe_words.reshape(B2 * W // 128, 128))
    return out[0, 0]


# speedup vs baseline: 1.1554x; 1.1554x over previous
import functools

import jax
import jax.numpy as jnp
from jax import lax
from jax.experimental import pallas as pl
from jax.experimental.pallas import tpu as pltpu
from jax.experimental.pallas import tpu_sc as plsc

VOCAB = 1000000
DIM = 64
DP = DIM // 2
B = 16384
NEG = 20
B2 = 4096
L = 5
W = 10
NC, NS = 2, 16
NW = NC * NS
CG = 64
NROWS = CG * NEG
NTN = NROWS // 128
NCW = B // NW // CG
NCM = (B2 * W) // NW // CG
B2W_ = B2 // NW

_ILV = plsc.PackFormat.INTERLEAVED


def _bc(s, n=16):
    return lax.broadcast_in_dim(s, (n,), ())


@functools.partial(
    pl.kernel,
    out_type=(jax.ShapeDtypeStruct((B * NEG,), jnp.float32),
              jax.ShapeDtypeStruct((B,), jnp.float32),
              jax.ShapeDtypeStruct((B2 * W * NEG,), jnp.float32),
              jax.ShapeDtypeStruct((B2 * W,), jnp.float32)),
    mesh=plsc.VectorSubcoreMesh(core_axis_name="c", subcore_axis_name="s"),
    compiler_params=pltpu.CompilerParams(
        use_tc_tiling_on_sc=False, needs_layout_passes=False),
    scratch_types=[
        pltpu.VMEM((2, NROWS, DP), jnp.int32),
        pltpu.VMEM((2, CG, DP), jnp.int32),
        pltpu.VMEM((2, CG, DP), jnp.int32),
        pltpu.VMEM((B2W_, DP), jnp.int32),
        pltpu.VMEM((2, NTN, 128), jnp.int32),
        pltpu.VMEM((2, CG), jnp.int32),
        pltpu.VMEM((2, CG), jnp.int32),
        pltpu.VMEM((2, NROWS), jnp.float32),
        pltpu.VMEM((NCM * CG,), jnp.float32),
        pltpu.VMEM((B2W_,), jnp.int32),
        pltpu.SemaphoreType.DMA,
        pltpu.SemaphoreType.DMA,
        pltpu.SemaphoreType.DMA,
    ],
)
def _sc_dots(ct, xt, cw2, ow2, nw2, mw3, ml2, om2, nm2,
             dnw_out, dpw_out, dnm_out, dpm_out,
             nvm, pvm, cvm, mvm, nidx, pidx, cidx, dnvm, dp_all, lvm,
             sem_i, sem_g, sem_w):
    wid = lax.axis_index("s") * NC + lax.axis_index("c")
    iota = lax.iota(jnp.int32, 16)

    def unpk(w):
        return plsc.unpack(plsc.bitcast(w, jnp.bfloat16), format=_ILV,
                           preferred_element_type=jnp.float32)

    pltpu.sync_copy(mw3.at[wid], nidx.at[0, pl.ds(0, L)])
    pltpu.sync_copy(ml2.at[wid], lvm)
    cps = [pltpu.async_copy(ct.at[nidx.at[0, t]],
                            nvm.at[0, pl.ds(t * 128, 128)], sem_g)
           for t in range(L)]
    for cp in cps:
        cp.wait()
    for bb in range(B2W_ // 16):
        bv = bb * 16 + iota
        lnv = lvm[pl.ds(bb * 16, 16)]
        lnf = lnv.astype(jnp.float32)

        def mbody(p, _):
            col = _bc(p)
            acc_e = jnp.zeros((16,), jnp.float32)
            acc_o = jnp.zeros((16,), jnp.float32)
            for l in range(L):
                re, ro = unpk(plsc.load_gather(nvm, [_bc(0), bv * L + l, col]))
                m = jnp.full((16,), l, jnp.int32) < lnv
                acc_e = acc_e + jnp.where(m, re, 0.0)
                acc_o = acc_o + jnp.where(m, ro, 0.0)
            packed = plsc.bitcast(
                plsc.pack(acc_e / lnf, acc_o / lnf, format=_ILV), jnp.int32)
            plsc.store_scatter(mvm, [bv, col], packed)
            return 0
        lax.fori_loop(0, DP, mbody, 0)

    def run_phase(ncc, is_word):
        cbase = wid * ncc

        def idx_copies(c, buf):
            cglob = cbase + c
            ops = [pltpu.make_async_copy(
                (nw2 if is_word else nm2).at[pl.ds(cglob * NTN, NTN)],
                nidx.at[buf], sem_i)]
            prow, pcol = cglob // 2, (cglob % 2) * CG
            ops.append(pltpu.make_async_copy(
                (ow2 if is_word else om2).at[prow, pl.ds(pcol, CG)],
                pidx.at[buf], sem_i))
            if is_word:
                ops.append(pltpu.make_async_copy(
                    cw2.at[prow, pl.ds(pcol, CG)], cidx.at[buf], sem_i))
            return ops

        def row_gathers(c, buf):
            ops = [pltpu.make_async_copy(
                xt.at[nidx.at[buf, t]],
                nvm.at[buf, pl.ds(t * 128, 128)], sem_g)
                for t in range(NTN)]
            ops.append(pltpu.make_async_copy(
                xt.at[pidx.at[buf]], pvm.at[buf], sem_g))
            if is_word:
                ops.append(pltpu.make_async_copy(
                    ct.at[cidx.at[buf]], cvm.at[buf], sem_g))
            return ops

        def dn_writeback(c, buf):
            cglob = cbase + c
            return pltpu.make_async_copy(
                dnvm.at[buf],
                (dnw_out if is_word else dnm_out).at[pl.ds(cglob * NROWS, NROWS)],
                sem_w)

        for op in idx_copies(0, 0):
            op.start()
            op.wait()
        if ncc > 1:
            for op in idx_copies(1, 1):
                op.start()
                op.wait()
        for op in row_gathers(0, 0):
            op.start()

        def chunk_body(c, _):
            buf = lax.rem(c, 2)
            nbuf = lax.rem(c + 1, 2)

            @pl.when((c >= 1) & (c + 1 < ncc))
            def _():
                for op in idx_copies(c + 1, nbuf):
                    op.wait()

            @pl.when(c + 1 < ncc)
            def _():
                for op in row_gathers(c + 1, nbuf):
                    op.start()

            for op in row_gathers(c, buf):
                op.wait()

            @pl.when(c + 2 < ncc)
            def _():
                for op in idx_copies(c + 2, buf):
                    op.start()

            @pl.when(c >= 2)
            def _():
                dn_writeback(c - 2, buf).wait()

            for kk in range(CG // 16):
                g = kk * 16 + iota
                if is_word:
                    crow = g
                else:
                    crow = (c * CG + g) // W
                nbase = g * NEG
                bufv = _bc(buf)

                def dbody(p, accs):
                    col = _bc(p)
                    if is_word:
                        ce, co = unpk(plsc.load_gather(cvm, [bufv, crow, col]))
                    else:
                        ce, co = unpk(plsc.load_gather(mvm, [crow, col]))
                    new = []
                    for j in range(NEG):
                        xe, xo = unpk(
                            plsc.load_gather(nvm, [bufv, nbase + j, col]))
                        new.append(accs[j] + xe * ce + xo * co)
                    pe, po = unpk(plsc.load_gather(pvm, [bufv, g, col]))
                    new.append(accs[NEG] + pe * ce + po * co)
                    return tuple(new)
                accs = lax.fori_loop(
                    0, DP, dbody,
                    tuple(jnp.zeros((16,), jnp.float32) for _ in range(NEG + 1)))
                if is_word:
                    for j in range(NEG):
                        plsc.store_scatter(dnvm, [bufv, nbase + j], accs[j])
                    plsc.store_scatter(dp_all, [c * CG + g], -accs[NEG])
                else:
                    kval = plsc.load_gather(pidx, [bufv, g])
                    keep = kval != 0
                    neg_big = jnp.full((16,), -1e9, jnp.float32)
                    for j in range(NEG):
                        v = jnp.where(keep, accs[j], neg_big)
                        plsc.store_scatter(dnvm, [bufv, nbase + j], v)
                    vp = jnp.where(keep, -accs[NEG], neg_big)
                    plsc.store_scatter(dp_all, [c * CG + g], vp)
            dn_writeback(c, buf).start()
            return 0
        lax.fori_loop(0, ncc, chunk_body, 0)

        if ncc >= 2:
            dn_writeback(ncc - 2, (ncc - 2) % 2).wait()
        dn_writeback(ncc - 1, (ncc - 1) % 2).wait()
        pw_out = dpw_out if is_word else dpm_out
        pltpu.sync_copy(dp_all.at[pl.ds(0, ncc * CG)],
                        pw_out.at[pl.ds(cbase * CG, ncc * CG)])

    run_phase(NCW, True)
    run_phase(NCM, False)


def _tc_body(dnw_ref, dpw_ref, dnm_ref, dpm_ref, omw_ref, out_ref):
    def sp_sum(x):
        return jnp.sum(jnp.maximum(x, 0.0) + jnp.log(1.0 + jnp.exp(-jnp.abs(x))))
    lw = sp_sum(dnw_ref[...]) + sp_sum(dpw_ref[...])
    lm = sp_sum(dnm_ref[...]) + sp_sum(dpm_ref[...])
    cnt = jnp.sum((omw_ref[...] != 0).astype(jnp.float32))
    out_ref[...] = jnp.reshape(lw / B + 25.0 * lm / cnt, (1, 1))


def _packed(table):
    bf = table.astype(jnp.bfloat16).reshape(VOCAB, DP, 2)
    return lax.bitcast_convert_type(bf, jnp.int32)


def kernel(center_words, outside_words, negative_examples_words, mwe_words,
           mwe_length, outside_mwe_words, negative_examples_mwe,
           center_table, context_table):
    cw2 = center_words.reshape(B // 128, 128)
    ow2 = outside_words.reshape(B // 128, 128)
    nw2 = negative_examples_words.reshape(B * NEG // 128, 128)
    mw3 = mwe_words.reshape(NW, L, 128)
    ml2 = mwe_length.reshape(NW, B2W_)
    om2 = outside_mwe_words.reshape(B2 * W // 128, 128)
    nm2 = negative_examples_mwe.reshape(B2 * W * NEG // 128, 128)

    dnw, dpw, dnm, dpm = _sc_dots(_packed(center_table), _packed(context_table),
                                  cw2, ow2, nw2, mw3, ml2, om2, nm2)

    out = pl.pallas_call(
        _tc_body,
        out_shape=jax.ShapeDtypeStruct((1, 1), jnp.float32),
    )(dnw.reshape(B * NEG // 128, 128),
      dpw.reshape(B // 128, 128),
      dnm.reshape(B2 * W * NEG // 128, 128),
      dpm.reshape(B2 * W // 128, 128),
      outside_mwe_words.reshape(B2 * W // 128, 128))
    return out[0, 0]


# speedup vs baseline: 3.5685x; 3.0885x over previous
import functools

import jax
import jax.numpy as jnp
from jax import lax
from jax.experimental import pallas as pl
from jax.experimental.pallas import tpu as pltpu
from jax.experimental.pallas import tpu_sc as plsc

VOCAB = 1000000
DIM = 64
B = 16384
NEG = 20
B2 = 4096
L = 5
W = 10
NC, NS = 2, 16
NW = NC * NS
CG = 32
NROWS = CG * NEG
NTN = NROWS // 128
NCW = B // NW // CG
NCM = (B2 * W) // NW // CG
B2W_ = B2 // NW


def _bc(s, n=16):
    return lax.broadcast_in_dim(s, (n,), ())


@functools.partial(
    pl.kernel,
    out_type=(jax.ShapeDtypeStruct((B * NEG,), jnp.float32),
              jax.ShapeDtypeStruct((B,), jnp.float32),
              jax.ShapeDtypeStruct((B2 * W * NEG,), jnp.float32),
              jax.ShapeDtypeStruct((B2 * W,), jnp.float32)),
    mesh=plsc.VectorSubcoreMesh(core_axis_name="c", subcore_axis_name="s"),
    compiler_params=pltpu.CompilerParams(
        use_tc_tiling_on_sc=False, needs_layout_passes=False),
    scratch_types=[
        pltpu.VMEM((2, NROWS, DIM), jnp.float32),
        pltpu.VMEM((2, CG, DIM), jnp.float32),
        pltpu.VMEM((2, CG, DIM), jnp.float32),
        pltpu.VMEM((B2W_, DIM), jnp.float32),
        pltpu.VMEM((2, NTN, 128), jnp.int32),
        pltpu.VMEM((2, CG), jnp.int32),
        pltpu.VMEM((2, CG), jnp.int32),
        pltpu.VMEM((2, NROWS), jnp.float32),
        pltpu.VMEM((NCM * CG,), jnp.float32),
        pltpu.VMEM((B2W_,), jnp.int32),
        pltpu.SemaphoreType.DMA,
        pltpu.SemaphoreType.DMA,
        pltpu.SemaphoreType.DMA,
    ],
)
def _sc_dots(ct, xt, cw2, ow2, nw2, mw3, ml2, om2, nm2,
             dnw_out, dpw_out, dnm_out, dpm_out,
             nvm, pvm, cvm, mvm, nidx, pidx, cidx, dnvm, dp_all, lvm,
             sem_i, sem_g, sem_w):
    wid = lax.axis_index("s") * NC + lax.axis_index("c")
    iota = lax.iota(jnp.int32, 16)

    pltpu.sync_copy(mw3.at[wid], nidx.at[0])
    pltpu.sync_copy(ml2.at[wid], lvm)
    cps = [pltpu.async_copy(ct.at[nidx.at[0, t]],
                            nvm.at[0, pl.ds(t * 128, 128)], sem_g)
           for t in range(L)]
    for cp in cps:
        cp.wait()
    for bb in range(B2W_ // 16):
        bv = bb * 16 + iota
        lnv = lvm[pl.ds(bb * 16, 16)]
        lnf = lnv.astype(jnp.float32)

        def mbody(d, _):
            col = (_bc(d) + iota) & (DIM - 1)
            acc = jnp.zeros((16,), jnp.float32)
            for l in range(L):
                r = plsc.load_gather(nvm, [_bc(0), bv * L + l, col])
                m = jnp.full((16,), l, jnp.int32) < lnv
                acc = acc + jnp.where(m, r, 0.0)
            plsc.store_scatter(mvm, [bv, col], acc / lnf)
            return 0
        lax.fori_loop(0, DIM, mbody, 0)

    def run_phase(ncc, is_word):
        cbase = wid * ncc

        def idx_copies(c, buf):
            cglob = cbase + c
            ops = [pltpu.make_async_copy(
                (nw2 if is_word else nm2).at[pl.ds(cglob * NTN, NTN)],
                nidx.at[buf], sem_i)]
            prow, pcol = cglob // 4, (cglob % 4) * CG
            ops.append(pltpu.make_async_copy(
                (ow2 if is_word else om2).at[prow, pl.ds(pcol, CG)],
                pidx.at[buf], sem_i))
            if is_word:
                ops.append(pltpu.make_async_copy(
                    cw2.at[prow, pl.ds(pcol, CG)], cidx.at[buf], sem_i))
            return ops

        def row_gathers(c, buf):
            ops = [pltpu.make_async_copy(
                xt.at[nidx.at[buf, t]],
                nvm.at[buf, pl.ds(t * 128, 128)], sem_g)
                for t in range(NTN)]
            ops.append(pltpu.make_async_copy(
                xt.at[pidx.at[buf]], pvm.at[buf], sem_g))
            if is_word:
                ops.append(pltpu.make_async_copy(
                    ct.at[cidx.at[buf]], cvm.at[buf], sem_g))
            return ops

        def dn_writeback(c, buf):
            cglob = cbase + c
            return pltpu.make_async_copy(
                dnvm.at[buf],
                (dnw_out if is_word else dnm_out).at[pl.ds(cglob * NROWS, NROWS)],
                sem_w)

        for op in idx_copies(0, 0):
            op.start()
            op.wait()
        if ncc > 1:
            for op in idx_copies(1, 1):
                op.start()
                op.wait()
        for op in row_gathers(0, 0):
            op.start()

        def chunk_body(c, _):
            buf = lax.rem(c, 2)
            nbuf = lax.rem(c + 1, 2)

            @pl.when((c >= 1) & (c + 1 < ncc))
            def _():
                for op in idx_copies(c + 1, nbuf):
                    op.wait()

            @pl.when(c + 1 < ncc)
            def _():
                for op in row_gathers(c + 1, nbuf):
                    op.start()

            for op in row_gathers(c, buf):
                op.wait()

            @pl.when(c + 2 < ncc)
            def _():
                for op in idx_copies(c + 2, buf):
                    op.start()

            @pl.when(c >= 2)
            def _():
                dn_writeback(c - 2, buf).wait()

            for kk in range(CG // 16):
                g = kk * 16 + iota
                if is_word:
                    crow = g
                else:
                    crow = (c * CG + g) // W
                nbase = g * NEG
                bufv = _bc(buf)

                def dbody(d, accs):
                    col = (_bc(d) + iota) & (DIM - 1)
                    if is_word:
                        cd = plsc.load_gather(cvm, [bufv, crow, col])
                    else:
                        cd = plsc.load_gather(mvm, [crow, col])
                    new = [
                        accs[j] + plsc.load_gather(nvm, [bufv, nbase + j, col]) * cd
                        for j in range(NEG)]
                    pd = plsc.load_gather(pvm, [bufv, g, col])
                    new.append(accs[NEG] + pd * cd)
                    return tuple(new)
                accs = lax.fori_loop(
                    0, DIM, dbody,
                    tuple(jnp.zeros((16,), jnp.float32) for _ in range(NEG + 1)))
                if is_word:
                    for j in range(NEG):
                        plsc.store_scatter(dnvm, [bufv, nbase + j], accs[j])
                    plsc.store_scatter(dp_all, [c * CG + g], -accs[NEG])
                else:
                    kval = plsc.load_gather(pidx, [bufv, g])
                    keep = kval != 0
                    neg_big = jnp.full((16,), -1e9, jnp.float32)
                    for j in range(NEG):
                        v = jnp.where(keep, accs[j], neg_big)
                        plsc.store_scatter(dnvm, [bufv, nbase + j], v)
                    vp = jnp.where(keep, -accs[NEG], neg_big)
                    plsc.store_scatter(dp_all, [c * CG + g], vp)
            dn_writeback(c, buf).start()
            return 0
        lax.fori_loop(0, ncc, chunk_body, 0)

        if ncc >= 2:
            dn_writeback(ncc - 2, (ncc - 2) % 2).wait()
        dn_writeback(ncc - 1, (ncc - 1) % 2).wait()
        pw_out = dpw_out if is_word else dpm_out
        pltpu.sync_copy(dp_all.at[pl.ds(0, ncc * CG)],
                        pw_out.at[pl.ds(cbase * CG, ncc * CG)])

    run_phase(NCW, True)
    run_phase(NCM, False)


def _tc_body(dnw_ref, dpw_ref, dnm_ref, dpm_ref, omw_ref, out_ref):
    def sp_sum(x):
        return jnp.sum(jnp.maximum(x, 0.0) + jnp.log(1.0 + jnp.exp(-jnp.abs(x))))
    lw = sp_sum(dnw_ref[...]) + sp_sum(dpw_ref[...])
    lm = sp_sum(dnm_ref[...]) + sp_sum(dpm_ref[...])
    cnt = jnp.sum((omw_ref[...] != 0).astype(jnp.float32))
    out_ref[...] = jnp.reshape(lw / B + 25.0 * lm / cnt, (1, 1))


def kernel(center_words, outside_words, negative_examples_words, mwe_words,
           mwe_length, outside_mwe_words, negative_examples_mwe,
           center_table, context_table):
    cw2 = center_words.reshape(B // 128, 128)
    ow2 = outside_words.reshape(B // 128, 128)
    nw2 = negative_examples_words.reshape(B * NEG // 128, 128)
    mw3 = mwe_words.reshape(NW, L, 128)
    ml2 = mwe_length.reshape(NW, B2W_)
    om2 = outside_mwe_words.reshape(B2 * W // 128, 128)
    nm2 = negative_examples_mwe.reshape(B2 * W * NEG // 128, 128)

    dnw, dpw, dnm, dpm = _sc_dots(center_table, context_table,
                                  cw2, ow2, nw2, mw3, ml2, om2, nm2)

    out = pl.pallas_call(
        _tc_body,
        out_shape=jax.ShapeDtypeStruct((1, 1), jnp.float32),
    )(dnw.reshape(B * NEG // 128, 128),
      dpw.reshape(B // 128, 128),
      dnm.reshape(B2 * W * NEG // 128, 128),
      dpm.reshape(B2 * W // 128, 128),
      outside_mwe_words.reshape(B2 * W // 128, 128))
    return out[0, 0]


# speedup vs baseline: 4.2929x; 1.2030x over previous
import functools

import jax
import jax.numpy as jnp
from jax import lax
from jax.experimental import pallas as pl
from jax.experimental.pallas import tpu as pltpu
from jax.experimental.pallas import tpu_sc as plsc

VOCAB = 1000000
DIM = 64
B = 16384
NEG = 20
B2 = 4096
L = 5
W = 10
NC, NS = 2, 16
NW = NC * NS
CG = 32
NROWS = CG * NEG
NTN = NROWS // 128
NCW = B // NW // CG
NCM = (B2 * W) // NW // CG
B2W_ = B2 // NW


def _bc(s, n=16):
    return lax.broadcast_in_dim(s, (n,), ())


@functools.partial(
    pl.kernel,
    out_type=(jax.ShapeDtypeStruct((B * NEG,), jnp.float32),
              jax.ShapeDtypeStruct((B,), jnp.float32),
              jax.ShapeDtypeStruct((B2 * W * NEG,), jnp.float32),
              jax.ShapeDtypeStruct((B2 * W,), jnp.float32)),
    mesh=plsc.VectorSubcoreMesh(core_axis_name="c", subcore_axis_name="s"),
    compiler_params=pltpu.CompilerParams(
        use_tc_tiling_on_sc=False, needs_layout_passes=False),
    scratch_types=[
        pltpu.VMEM((2, NROWS, DIM), jnp.float32),
        pltpu.VMEM((2, CG, DIM), jnp.float32),
        pltpu.VMEM((2, CG, DIM), jnp.float32),
        pltpu.VMEM((B2W_, DIM), jnp.float32),
        pltpu.VMEM((2, NTN, 128), jnp.int32),
        pltpu.VMEM((2, CG), jnp.int32),
        pltpu.VMEM((2, NROWS), jnp.float32),
        pltpu.VMEM((NCM * CG,), jnp.float32),
        pltpu.VMEM((B2W_,), jnp.int32),
        pltpu.SemaphoreType.DMA,
        pltpu.SemaphoreType.DMA,
        pltpu.SemaphoreType.DMA,
    ],
)
def _sc_dots(cr, mr, xt, ow2, nw2, ml2, om2, nm2,
             dnw_out, dpw_out, dnm_out, dpm_out,
             nvm, pvm, cvm, mvm, nidx, pidx, dnvm, dp_all, lvm,
             sem_i, sem_g, sem_w):
    wid = lax.axis_index("s") * NC + lax.axis_index("c")
    iota = lax.iota(jnp.int32, 16)

    pltpu.sync_copy(ml2.at[wid], lvm)
    pltpu.sync_copy(mr.at[pl.ds(wid * B2W_ * L, B2W_ * L)],
                    nvm.at[0, pl.ds(0, B2W_ * L)])
    for bb in range(B2W_ // 16):
        bv = bb * 16 + iota
        lnv = lvm[pl.ds(bb * 16, 16)]
        lnf = lnv.astype(jnp.float32)

        def mbody(d, _):
            col = (_bc(d) + iota) & (DIM - 1)
            acc = jnp.zeros((16,), jnp.float32)
            for l in range(L):
                r = plsc.load_gather(nvm, [_bc(0), bv * L + l, col])
                m = jnp.full((16,), l, jnp.int32) < lnv
                acc = acc + jnp.where(m, r, 0.0)
            plsc.store_scatter(mvm, [bv, col], acc / lnf)
            return 0
        lax.fori_loop(0, DIM, mbody, 0)

    def run_phase(ncc, is_word):
        cbase = wid * ncc

        def idx_copies(c, buf):
            cglob = cbase + c
            ops = [pltpu.make_async_copy(
                (nw2 if is_word else nm2).at[pl.ds(cglob * NTN, NTN)],
                nidx.at[buf], sem_i)]
            prow, pcol = cglob // 4, (cglob % 4) * CG
            ops.append(pltpu.make_async_copy(
                (ow2 if is_word else om2).at[prow, pl.ds(pcol, CG)],
                pidx.at[buf], sem_i))
            return ops

        def row_gathers(c, buf):
            ops = [pltpu.make_async_copy(
                xt.at[nidx.at[buf, t]],
                nvm.at[buf, pl.ds(t * 128, 128)], sem_g)
                for t in range(NTN)]
            ops.append(pltpu.make_async_copy(
                xt.at[pidx.at[buf]], pvm.at[buf], sem_g))
            if is_word:
                ops.append(pltpu.make_async_copy(
                    cr.at[pl.ds((cbase + c) * CG, CG)], cvm.at[buf], sem_g))
            return ops

        def dn_writeback(c, buf):
            cglob = cbase + c
            return pltpu.make_async_copy(
                dnvm.at[buf],
                (dnw_out if is_word else dnm_out).at[pl.ds(cglob * NROWS, NROWS)],
                sem_w)

        for op in idx_copies(0, 0):
            op.start()
            op.wait()
        if ncc > 1:
            for op in idx_copies(1, 1):
                op.start()
                op.wait()
        for op in row_gathers(0, 0):
            op.start()

        def chunk_body(c, _):
            buf = lax.rem(c, 2)
            nbuf = lax.rem(c + 1, 2)

            @pl.when((c >= 1) & (c + 1 < ncc))
            def _():
                for op in idx_copies(c + 1, nbuf):
                    op.wait()

            @pl.when(c + 1 < ncc)
            def _():
                for op in row_gathers(c + 1, nbuf):
                    op.start()

            for op in row_gathers(c, buf):
                op.wait()

            @pl.when(c + 2 < ncc)
            def _():
                for op in idx_copies(c + 2, buf):
                    op.start()

            @pl.when(c >= 2)
            def _():
                dn_writeback(c - 2, buf).wait()

            for kk in range(CG // 16):
                g = kk * 16 + iota
                if is_word:
                    crow = g
                else:
                    crow = (c * CG + g) // W
                nbase = g * NEG
                bufv = _bc(buf)

                def dbody(d, accs):
                    col = (_bc(d) + iota) & (DIM - 1)
                    if is_word:
                        cd = plsc.load_gather(cvm, [bufv, crow, col])
                    else:
                        cd = plsc.load_gather(mvm, [crow, col])
                    new = [
                        accs[j] + plsc.load_gather(nvm, [bufv, nbase + j, col]) * cd
                        for j in range(NEG)]
                    pd = plsc.load_gather(pvm, [bufv, g, col])
                    new.append(accs[NEG] + pd * cd)
                    return tuple(new)
                accs = lax.fori_loop(
                    0, DIM, dbody,
                    tuple(jnp.zeros((16,), jnp.float32) for _ in range(NEG + 1)))
                if is_word:
                    for j in range(NEG):
                        plsc.store_scatter(dnvm, [bufv, nbase + j], accs[j])
                    plsc.store_scatter(dp_all, [c * CG + g], -accs[NEG])
                else:
                    kval = plsc.load_gather(pidx, [bufv, g])
                    keep = kval != 0
                    neg_big = jnp.full((16,), -1e9, jnp.float32)
                    for j in range(NEG):
                        v = jnp.where(keep, accs[j], neg_big)
                        plsc.store_scatter(dnvm, [bufv, nbase + j], v)
                    vp = jnp.where(keep, -accs[NEG], neg_big)
                    plsc.store_scatter(dp_all, [c * CG + g], vp)
            dn_writeback(c, buf).start()
            return 0
        lax.fori_loop(0, ncc, chunk_body, 0)

        if ncc >= 2:
            dn_writeback(ncc - 2, (ncc - 2) % 2).wait()
        dn_writeback(ncc - 1, (ncc - 1) % 2).wait()
        pw_out = dpw_out if is_word else dpm_out
        pltpu.sync_copy(dp_all.at[pl.ds(0, ncc * CG)],
                        pw_out.at[pl.ds(cbase * CG, ncc * CG)])

    run_phase(NCW, True)
    run_phase(NCM, False)


def _tc_body(dnw_ref, dpw_ref, dnm_ref, dpm_ref, omw_ref, out_ref):
    def sp_sum(x):
        return jnp.sum(jnp.maximum(x, 0.0) + jnp.log(1.0 + jnp.exp(-jnp.abs(x))))
    lw = sp_sum(dnw_ref[...]) + sp_sum(dpw_ref[...])
    lm = sp_sum(dnm_ref[...]) + sp_sum(dpm_ref[...])
    cnt = jnp.sum((omw_ref[...] != 0).astype(jnp.float32))
    out_ref[...] = jnp.reshape(lw / B + 25.0 * lm / cnt, (1, 1))


def kernel(center_words, outside_words, negative_examples_words, mwe_words,
           mwe_length, outside_mwe_words, negative_examples_mwe,
           center_table, context_table):
    ow2 = outside_words.reshape(B // 128, 128)
    nw2 = negative_examples_words.reshape(B * NEG // 128, 128)
    ml2 = mwe_length.reshape(NW, B2W_)
    om2 = outside_mwe_words.reshape(B2 * W // 128, 128)
    nm2 = negative_examples_mwe.reshape(B2 * W * NEG // 128, 128)

    crows = jnp.take(center_table, center_words, axis=0)
    mrows = jnp.take(center_table, mwe_words.reshape(-1), axis=0)

    dnw, dpw, dnm, dpm = _sc_dots(crows, mrows, context_table,
                                  ow2, nw2, ml2, om2, nm2)

    out = pl.pallas_call(
        _tc_body,
        out_shape=jax.ShapeDtypeStruct((1, 1), jnp.float32),
    )(dnw.reshape(B * NEG // 128, 128),
      dpw.reshape(B // 128, 128),
      dnm.reshape(B2 * W * NEG // 128, 128),
      dpm.reshape(B2 * W // 128, 128),
      outside_mwe_words.reshape(B2 * W // 128, 128))
    return out[0, 0]
